# baseline (speedup 1.0000x reference)
import functools

import jax
import jax.numpy as jnp
from jax import lax
from jax.experimental import pallas as pl
from jax.experimental.pallas import tpu as pltpu
from jax.experimental.pallas import tpu_sc as plsc

_LANES = 16
_GI = 128


def _make_sc_gather_pe(n_rows, hidden, seq_len, chunk, n_workers, nc):
    rows_per_w = n_rows // n_workers
    n_chunks = rows_per_w // chunk
    sub = chunk // _GI
    cvecs = hidden // _LANES

    mesh = plsc.VectorSubcoreMesh(core_axis_name="c", subcore_axis_name="s")

    @functools.partial(
        pl.kernel,
        out_type=jax.ShapeDtypeStruct((n_rows, hidden), jnp.float32),
        mesh=mesh,
        compiler_params=pltpu.CompilerParams(use_tc_tiling_on_sc=False),
        scratch_types=[
            pltpu.VMEM((seq_len, hidden), jnp.float32),
            pltpu.VMEM((chunk,), jnp.int32),
            pltpu.VMEM((chunk,), jnp.int32),
            pltpu.VMEM((chunk, hidden), jnp.float32),
            pltpu.VMEM((chunk, hidden), jnp.float32),
            pltpu.SemaphoreType.DMA,
            pltpu.SemaphoreType.DMA,
            pltpu.SemaphoreType.DMA,
            pltpu.SemaphoreType.DMA,
            pltpu.SemaphoreType.DMA,
            pltpu.SemaphoreType.DMA,
        ],
    )
    def gather_pe(table, idxf, pe, out, pe_v, idx0, idx1, rows0, rows1,
                  isem0, isem1, gsem0, gsem1, ssem0, ssem1):
        idxs = (idx0, idx1)
        rows = (rows0, rows1)
        isems = (isem0, isem1)
        gsems = (gsem0, gsem1)
        ssems = (ssem0, ssem1)

        wid = lax.axis_index("s") * nc + lax.axis_index("c")
        base = wid * rows_per_w

        pltpu.sync_copy(pe, pe_v)

        def idx_copy(g, b):
            pltpu.async_copy(idxf.at[pl.ds(base + g * chunk, chunk)],
                             idxs[b], isems[b])

        def idx_wait(b):
            pltpu.make_async_copy(idxf.at[pl.ds(0, chunk)],
                                  idxs[b], isems[b]).wait()

        def gather(b):
            for i in range(sub):
                pltpu.async_copy(
                    table.at[idxs[b].at[pl.ds(i * _GI, _GI)]],
                    rows[b].at[pl.ds(i * _GI, _GI)], gsems[b])

        def gather_wait(b):
            pltpu.make_async_copy(table.at[pl.ds(0, chunk)],
                                  rows[b], gsems[b]).wait()

        def store(g, b):
            pltpu.async_copy(rows[b],
                             out.at[pl.ds(base + g * chunk, chunk)], ssems[b])

        def store_wait(b):
            pltpu.make_async_copy(rows[b],
                                  out.at[pl.ds(0, chunk)], ssems[b]).wait()

        def add_pe(g, b):
            pe_row0 = lax.rem(g * chunk, seq_len)

            def row_body(r, pr):
                for c in range(cvecs):
                    sl = pl.ds(c * _LANES, _LANES)
                    rows[b][r, sl] = rows[b][r, sl] + pe_v[pr, sl]
                pr = pr + 1
                return jnp.where(pr == seq_len, 0, pr)

            lax.fori_loop(0, chunk, row_body, pe_row0, unroll=2)

        idx_copy(0, 0)
        idx_copy(1, 1)
        idx_wait(0)
        gather(0)

        def pair_step(i, _):
            for b in (0, 1):
                g = 2 * i + b
                nb = 1 - b

                @pl.when(g + 1 < n_chunks)
                def _():
                    idx_wait(nb)

                    @pl.when(g >= 1)
                    def _():
                        store_wait(nb)

                    gather(nb)

                gather_wait(b)

                @pl.when(g + 2 < n_chunks)
                def _():
                    idx_copy(g + 2, b)

                add_pe(g, b)
                store(g, b)
            return 0

        lax.fori_loop(0, n_chunks // 2, pair_step, 0)
        store_wait(0)
        store_wait(1)

    return gather_pe


def kernel(x, table, pe):
    batch, seq = x.shape
    _, hidden = table.shape
    n_rows = batch * seq

    n_workers = 32
    nc = 2
    chunk = 512

    idxf = x.astype(jnp.int32).reshape(n_rows)
    pe2 = pe[0, :seq, :]

    fn = _make_sc_gather_pe(n_rows, hidden, seq, chunk, n_workers, nc)
    out = fn(table, idxf, pe2)
    return out.reshape(batch, seq, hidden)


# speedup vs baseline: 1.3052x; 1.3052x over previous
import functools

import jax
import jax.numpy as jnp
from jax import lax
from jax.experimental import pallas as pl
from jax.experimental.pallas import tpu as pltpu
from jax.experimental.pallas import tpu_sc as plsc

_GI = 128
_NBUF = 4


def _make_sc_gather_pe(n_rows, hidden, seq_len, chunk, n_workers, nc):
    rows_per_w = n_rows // n_workers
    n_chunks = rows_per_w // chunk
    assert n_chunks % _NBUF == 0
    assert chunk % seq_len == 0
    reps = chunk // seq_len
    splits = []
    s = 0
    while s < chunk:
        splits.append((s, min(_GI, chunk - s)))
        s += _GI

    mesh = plsc.VectorSubcoreMesh(core_axis_name="c", subcore_axis_name="s")

    @functools.partial(
        pl.kernel,
        out_type=jax.ShapeDtypeStruct((n_rows, hidden), jnp.float32),
        mesh=mesh,
        compiler_params=pltpu.CompilerParams(use_tc_tiling_on_sc=False),
        scratch_types=[
            pltpu.VMEM_SHARED((seq_len, hidden), jnp.float32),
            tuple(pltpu.VMEM((chunk,), jnp.int32) for _ in range(_NBUF)),
            tuple(pltpu.VMEM((chunk, hidden), jnp.float32) for _ in range(_NBUF)),
            tuple(pltpu.SemaphoreType.DMA for _ in range(_NBUF)),
            tuple(pltpu.SemaphoreType.DMA for _ in range(_NBUF)),
            tuple(pltpu.SemaphoreType.DMA for _ in range(_NBUF)),
            tuple(pltpu.SemaphoreType.DMA for _ in range(_NBUF)),
        ],
    )
    def gather_pe(table, idxf, pe, out, pe_sh, idxs, rows,
                  isems, psems, gsems, ssems):
        wid = lax.axis_index("s") * nc + lax.axis_index("c")
        base = wid * rows_per_w

        @pl.when(lax.axis_index("s") == 0)
        def _():
            pltpu.sync_copy(pe, pe_sh)
        plsc.subcore_barrier()

        def idx_copy(g, b):
            pltpu.async_copy(idxf.at[pl.ds(base + g * chunk, chunk)],
                             idxs[b], isems[b])

        def idx_wait(b):
            pltpu.make_async_copy(idxf.at[pl.ds(0, chunk)],
                                  idxs[b], isems[b]).wait()

        def prefill(b):
            for r in range(reps):
                pltpu.async_copy(
                    pe_sh, rows[b].at[pl.ds(r * seq_len, seq_len)], psems[b])

        def prefill_wait(b):
            pltpu.make_async_copy(table.at[pl.ds(0, chunk)],
                                  rows[b], psems[b]).wait()

        def gather_add(b):
            for (s0, sz) in splits:
                pltpu.async_copy(
                    table.at[idxs[b].at[pl.ds(s0, sz)]],
                    rows[b].at[pl.ds(s0, sz)], gsems[b], add=True)

        def gather_wait(b):
            pltpu.make_async_copy(table.at[pl.ds(0, chunk)],
                                  rows[b], gsems[b]).wait()

        def store(g, b):
            pltpu.async_copy(rows[b],
                             out.at[pl.ds(base + g * chunk, chunk)], ssems[b])

        def store_wait(b):
            pltpu.make_async_copy(rows[b],
                                  out.at[pl.ds(0, chunk)], ssems[b]).wait()

        def prep(g, b, drain_store):
            if drain_store:
                @pl.when(g >= _NBUF)
                def _():
                    store_wait(b)
            prefill(b)
            idx_wait(b)
            prefill_wait(b)
            gather_add(b)

        for b in range(_NBUF):
            idx_copy(b, b)
        for g in range(_NBUF - 1):
            prep(g, g, drain_store=False)

        def ring_step(i, _):
            for b in range(_NBUF):
                g = _NBUF * i + b
                gp = g + _NBUF - 1

                pb = (b + _NBUF - 1) % _NBUF

                @pl.when(gp < n_chunks)
                def _():
                    prep(gp, pb, drain_store=True)

                gather_wait(b)

                @pl.when(g + _NBUF < n_chunks)
                def _():
                    idx_copy(g + _NBUF, b)

                store(g, b)
            return 0

        lax.fori_loop(0, n_chunks // _NBUF, ring_step, 0)
        for b in range(_NBUF):
            store_wait(b)

    return gather_pe


def kernel(x, table, pe):
    batch, seq = x.shape
    _, hidden = table.shape
    n_rows = batch * seq

    n_workers = 32
    nc = 2
    chunk = 2 * seq

    idxf = x.astype(jnp.int32).reshape(n_rows)
    pe2 = pe[0, :seq, :]

    fn = _make_sc_gather_pe(n_rows, hidden, seq, chunk, n_workers, nc)
    out = fn(table, idxf, pe2)
    return out.reshape(batch, seq, hidden)


# speedup vs baseline: 1.3155x; 1.0079x over previous
import functools

import jax
import jax.numpy as jnp
from jax import lax
from jax.experimental import pallas as pl
from jax.experimental.pallas import tpu as pltpu
from jax.experimental.pallas import tpu_sc as plsc

_GI = 128
_NBUF = 4


def _make_sc_gather_pe(n_rows, hidden, seq_len, chunk, n_workers, nc):
    rows_per_w = n_rows // n_workers
    n_chunks = rows_per_w // chunk
    assert n_chunks % _NBUF == 0
    assert chunk % seq_len == 0
    reps = chunk // seq_len
    splits = []
    s = 0
    while s < chunk:
        splits.append((s, min(_GI, chunk - s)))
        s += _GI

    mesh = plsc.VectorSubcoreMesh(core_axis_name="c", subcore_axis_name="s")

    @functools.partial(
        pl.kernel,
        out_type=jax.ShapeDtypeStruct((n_rows, hidden), jnp.float32),
        mesh=mesh,
        compiler_params=pltpu.CompilerParams(use_tc_tiling_on_sc=False),
        scratch_types=[
            pltpu.VMEM((seq_len, hidden), jnp.float32),
            tuple(pltpu.VMEM((chunk,), jnp.int32) for _ in range(_NBUF)),
            tuple(pltpu.VMEM((chunk, hidden), jnp.float32) for _ in range(_NBUF)),
            tuple(pltpu.SemaphoreType.DMA for _ in range(_NBUF)),
            tuple(pltpu.SemaphoreType.DMA for _ in range(_NBUF)),
            tuple(pltpu.SemaphoreType.DMA for _ in range(_NBUF)),
        ],
    )
    def gather_pe(table, idxf, pe, out, pe_v, idxs, rows,
                  isems, gsems, ssems):
        wid = lax.axis_index("s") * nc + lax.axis_index("c")
        base = wid * rows_per_w

        pltpu.sync_copy(pe, pe_v)

        def idx_copy(g, b):
            pltpu.async_copy(idxf.at[pl.ds(base + g * chunk, chunk)],
                             idxs[b], isems[b])

        def idx_wait(b):
            pltpu.make_async_copy(idxf.at[pl.ds(0, chunk)],
                                  idxs[b], isems[b]).wait()

        def prefill(b):
            @plsc.parallel_loop(0, seq_len, unroll=8)
            def _(p):
                for c in range(hidden // 16):
                    sl = pl.ds(c * 16, 16)
                    v = pe_v[p, sl]
                    for r in range(reps):
                        rows[b][p + r * seq_len, sl] = v

        def gather_add(b):
            for (s0, sz) in splits:
                pltpu.async_copy(
                    table.at[idxs[b].at[pl.ds(s0, sz)]],
                    rows[b].at[pl.ds(s0, sz)], gsems[b], add=True)

        def gather_wait(b):
            pltpu.make_async_copy(table.at[pl.ds(0, chunk)],
                                  rows[b], gsems[b]).wait()

        def store(g, b):
            pltpu.async_copy(rows[b],
                             out.at[pl.ds(base + g * chunk, chunk)], ssems[b])

        def store_wait(b):
            pltpu.make_async_copy(rows[b],
                                  out.at[pl.ds(0, chunk)], ssems[b]).wait()

        def prep(g, b, drain_store):
            if drain_store:
                @pl.when(g >= _NBUF)
                def _():
                    store_wait(b)
            prefill(b)
            idx_wait(b)
            gather_add(b)

        for b in range(_NBUF):
            idx_copy(b, b)
        for g in range(_NBUF - 1):
            prep(g, g, drain_store=False)

        def ring_step(i, _):
            for b in range(_NBUF):
                g = _NBUF * i + b
                gp = g + _NBUF - 1

                pb = (b + _NBUF - 1) % _NBUF

                @pl.when(gp < n_chunks)
                def _():
                    prep(gp, pb, drain_store=True)

                gather_wait(b)

                @pl.when(g + _NBUF < n_chunks)
                def _():
                    idx_copy(g + _NBUF, b)

                store(g, b)
            return 0

        lax.fori_loop(0, n_chunks // _NBUF, ring_step, 0)
        for b in range(_NBUF):
            store_wait(b)

    return gather_pe


def kernel(x, table, pe):
    batch, seq = x.shape
    _, hidden = table.shape
    n_rows = batch * seq

    n_workers = 32
    nc = 2
    chunk = 2 * seq

    idxf = x.astype(jnp.int32).reshape(n_rows)
    pe2 = pe[0, :seq, :]

    fn = _make_sc_gather_pe(n_rows, hidden, seq, chunk, n_workers, nc)
    out = fn(table, idxf, pe2)
    return out.reshape(batch, seq, hidden)


# speedup vs baseline: 1.3165x; 1.0008x over previous
import functools

import jax
import jax.numpy as jnp
from jax import lax
from jax.experimental import pallas as pl
from jax.experimental.pallas import tpu as pltpu
from jax.experimental.pallas import tpu_sc as plsc

_GI = 128
_NBUF = 4


def _make_sc_gather_pe(n_rows, hidden, seq_len, chunk, n_workers, nc):
    rows_per_w = n_rows // n_workers
    n_chunks = rows_per_w // chunk
    assert n_chunks % _NBUF == 0
    assert chunk % seq_len == 0
    reps = chunk // seq_len
    splits = []
    s = 0
    while s < chunk:
        splits.append((s, min(_GI, chunk - s)))
        s += _GI

    mesh = plsc.VectorSubcoreMesh(core_axis_name="c", subcore_axis_name="s")

    @functools.partial(
        pl.kernel,
        out_type=jax.ShapeDtypeStruct((n_rows, hidden), jnp.float32),
        mesh=mesh,
        compiler_params=pltpu.CompilerParams(use_tc_tiling_on_sc=False),
        scratch_types=[
            pltpu.VMEM((seq_len, hidden), jnp.float32),
            tuple(pltpu.VMEM((chunk,), jnp.int32) for _ in range(_NBUF)),
            tuple(pltpu.VMEM((chunk, hidden), jnp.float32) for _ in range(_NBUF)),
            tuple(pltpu.SemaphoreType.DMA for _ in range(_NBUF)),
            tuple(pltpu.SemaphoreType.DMA for _ in range(_NBUF)),
            tuple(pltpu.SemaphoreType.DMA for _ in range(_NBUF)),
        ],
    )
    def gather_pe(table, idxf, pe, out, pe_v, idxs, rows,
                  isems, gsems, ssems):
        wid = lax.axis_index("s") * nc + lax.axis_index("c")
        base = wid * rows_per_w

        pltpu.sync_copy(pe, pe_v)

        def idx_copy(g, b):
            pltpu.async_copy(idxf.at[pl.ds(base + g * chunk, chunk)],
                             idxs[b], isems[b])

        def idx_wait(b):
            pltpu.make_async_copy(idxf.at[pl.ds(0, chunk)],
                                  idxs[b], isems[b]).wait()

        def prefill(b):
            @plsc.parallel_loop(0, seq_len, unroll=8)
            def _(p):
                for c in range(hidden // 16):
                    sl = pl.ds(c * 16, 16)
                    v = pe_v[p, sl]
                    for r in range(reps):
                        rows[b][p + r * seq_len, sl] = v

        def gather_add(b):
            pltpu.async_copy(table.at[idxs[b]], rows[b], gsems[b], add=True)

        def gather_wait(b):
            pltpu.make_async_copy(table.at[pl.ds(0, chunk)],
                                  rows[b], gsems[b]).wait()

        def store(g, b):
            pltpu.async_copy(rows[b],
                             out.at[pl.ds(base + g * chunk, chunk)], ssems[b])

        def store_wait(b):
            pltpu.make_async_copy(rows[b],
                                  out.at[pl.ds(0, chunk)], ssems[b]).wait()

        def prep(g, b, drain_store):
            if drain_store:
                @pl.when(g >= _NBUF)
                def _():
                    store_wait(b)
            prefill(b)
            idx_wait(b)
            gather_add(b)

        for b in range(_NBUF):
            idx_copy(b, b)
        for g in range(_NBUF - 1):
            prep(g, g, drain_store=False)

        def ring_step(i, _):
            for b in range(_NBUF):
                g = _NBUF * i + b
                gp = g + _NBUF - 1

                pb = (b + _NBUF - 1) % _NBUF

                @pl.when(gp < n_chunks)
                def _():
                    prep(gp, pb, drain_store=True)

                gather_wait(b)

                @pl.when(g + _NBUF < n_chunks)
                def _():
                    idx_copy(g + _NBUF, b)

                store(g, b)
            return 0

        lax.fori_loop(0, n_chunks // _NBUF, ring_step, 0)
        for b in range(_NBUF):
            store_wait(b)

    return gather_pe


def kernel(x, table, pe):
    batch, seq = x.shape
    _, hidden = table.shape
    n_rows = batch * seq

    n_workers = 32
    nc = 2
    chunk = 2 * seq

    idxf = x.astype(jnp.int32).reshape(n_rows)
    pe2 = pe[0, :seq, :]

    fn = _make_sc_gather_pe(n_rows, hidden, seq, chunk, n_workers, nc)
    out = fn(table, idxf, pe2)
    return out.reshape(batch, seq, hidden)
